# baseline (speedup 1.0000x reference)
import functools

import jax
import jax.numpy as jnp
from jax import lax
from jax.experimental import pallas as pl
from jax.experimental.pallas import tpu as pltpu
from jax.experimental.pallas import tpu_sc as plsc

N = 32768
DIM = 512
NCLS = 2
K = 6553

IBLK = 1024
JBLK = 2048

NUM_WORKERS = 32
CHUNK = N // NUM_WORKERS
GRP = 128


def _mm_body(h_ref, w_ref, b_ref, proj_ref, sm_ref):
    p = jnp.dot(h_ref[...], w_ref[...], preferred_element_type=jnp.float32)
    p = p + b_ref[...]
    proj_ref[...] = p
    m = jnp.max(p, axis=1, keepdims=True)
    e = jnp.exp(p - m)
    sm_ref[...] = e / jnp.sum(e, axis=1, keepdims=True)


def _proj_softmax(h, w, b):
    return pl.pallas_call(
        _mm_body,
        grid=(N // IBLK,),
        in_specs=[
            pl.BlockSpec((IBLK, DIM), lambda i: (i, 0)),
            pl.BlockSpec((DIM, NCLS), lambda i: (0, 0)),
            pl.BlockSpec((1, NCLS), lambda i: (0, 0)),
        ],
        out_specs=[
            pl.BlockSpec((IBLK, NCLS), lambda i: (i, 0)),
            pl.BlockSpec((IBLK, NCLS), lambda i: (i, 0)),
        ],
        out_shape=[
            jax.ShapeDtypeStruct((N, NCLS), jnp.float32),
            jax.ShapeDtypeStruct((N, NCLS), jnp.float32),
        ],
    )(h, w, b)


def _rank_body(si_ref, sall_ref, pos_ref, neg_ref):
    i = pl.program_id(0)
    si = si_ref[...]
    ii = i * IBLK + lax.broadcasted_iota(jnp.int32, (IBLK, 1), 0)

    def step(c, carry):
        g_acc, e_acc, b_acc = carry
        sj = sall_ref[:, pl.ds(c * JBLK, JBLK)]
        jj = c * JBLK + lax.broadcasted_iota(jnp.int32, (1, JBLK), 1)
        gt = (sj > si)
        eq = (sj == si)
        blt = eq & (jj < ii)
        g_acc = g_acc + jnp.sum(gt.astype(jnp.int32), axis=1, keepdims=True)
        e_acc = e_acc + jnp.sum(eq.astype(jnp.int32), axis=1, keepdims=True)
        b_acc = b_acc + jnp.sum(blt.astype(jnp.int32), axis=1, keepdims=True)
        return g_acc, e_acc, b_acc

    zero = jnp.zeros((IBLK, 1), jnp.int32)
    g, e, bb = lax.fori_loop(0, N // JBLK, step, (zero, zero, zero))
    col = lax.broadcasted_iota(jnp.int32, (IBLK, 4), 1)
    pos_ref[...] = 4 * (g + bb) + col
    neg_ref[...] = 4 * (N - g - e + bb) + col


def _ranks(scores_col, scores_row):
    return pl.pallas_call(
        _rank_body,
        grid=(N // IBLK,),
        in_specs=[
            pl.BlockSpec((IBLK, 1), lambda i: (i, 0)),
            pl.BlockSpec((1, N), lambda i: (0, 0)),
        ],
        out_specs=[
            pl.BlockSpec((IBLK, 4), lambda i: (i, 0)),
            pl.BlockSpec((IBLK, 4), lambda i: (i, 0)),
        ],
        out_shape=[
            jax.ShapeDtypeStruct((N, 4), jnp.int32),
            jax.ShapeDtypeStruct((N, 4), jnp.int32),
        ],
    )(scores_col, scores_row)


@functools.lru_cache(maxsize=None)
def _make_scatter():
    mesh = plsc.VectorSubcoreMesh(core_axis_name="c", subcore_axis_name="s")
    cw = CHUNK * 4
    ng = cw // GRP

    @functools.partial(
        pl.kernel,
        out_type=[
            jax.ShapeDtypeStruct((N * 4,), jnp.float32),
            jax.ShapeDtypeStruct((N * 4,), jnp.float32),
        ],
        mesh=mesh,
        scratch_types=[
            pltpu.VMEM((cw,), jnp.float32),
            pltpu.VMEM((ng, GRP), jnp.int32),
            pltpu.VMEM((ng, GRP), jnp.int32),
            pltpu.SemaphoreType.DMA,
        ],
    )
    def scatter_k(pay_hbm, pr_hbm, nr_hbm, outp_hbm, outn_hbm,
                  pay_v, pidx_v, nidx_v, sem):
        wid = lax.axis_index("s") * 2 + lax.axis_index("c")
        pltpu.sync_copy(pay_hbm.at[pl.ds(wid * cw, cw)], pay_v)
        pltpu.sync_copy(pr_hbm.at[wid], pidx_v)
        pltpu.sync_copy(nr_hbm.at[wid], nidx_v)
        copies = []
        for g in range(ng):
            src = pay_v.at[pl.ds(g * GRP, GRP)]
            copies.append(
                pltpu.async_copy(src, outp_hbm.at[pidx_v.at[g]], sem))
            copies.append(
                pltpu.async_copy(src, outn_hbm.at[nidx_v.at[g]], sem))
        for cp in copies:
            cp.wait()

    return scatter_k


def kernel(bag_label, h, A, W, b):
    scores = lax.dynamic_index_in_dim(A, bag_label, axis=2, keepdims=False)
    scores = scores[:, 0]
    scores_col = scores.reshape(N, 1)
    scores_row = scores.reshape(1, N)

    proj, sm = _proj_softmax(h, W, b.reshape(1, NCLS))
    posrank, negrank = _ranks(scores_col, scores_row)

    payload = jnp.concatenate([proj, sm], axis=1).reshape(N * 4)
    pr = posrank.reshape(NUM_WORKERS, CHUNK * 4 // GRP, GRP)
    nr = negrank.reshape(NUM_WORKERS, CHUNK * 4 // GRP, GRP)
    outp, outn = _make_scatter()(payload, pr, nr)
    outp = outp.reshape(N, 4)
    outn = outn.reshape(N, 4)

    un = jnp.concatenate([outp[:K, :2], outn[:K, :2]], axis=0)
    smx = jnp.concatenate([outp[:K, 2:4], outn[:K, 2:4]], axis=0)
    labels = jnp.concatenate(
        [jnp.ones((K,), jnp.int32), jnp.zeros((K,), jnp.int32)])
    return (labels, un, smx)
